# Initial kernel scaffold; baseline (speedup 1.0000x reference)
import jax
import jax.numpy as jnp
from jax.experimental import pallas as pl


def kernel(x, edge_index, batch, params):
    raise NotImplementedError("write your pallas kernel here")


# speedup vs baseline: 16.6744x; 16.6744x over previous
import functools
import math

import jax
import jax.numpy as jnp
from jax import lax
from jax.experimental import pallas as pl
from jax.experimental.pallas import tpu as pltpu
from jax.experimental.pallas import tpu_sc as plsc

N = 10000
E = 320000
NUM_GRAPHS = 50
NPG = 200
HID = 64
NUM_HEADS = 4
HD = HID // NUM_HEADS
NUM_SEEDS = 50
SCALE = 1.0 / math.sqrt(HID)

NC = 2
NS = 16
NW = NC * NS
EW = E // NW
CHUNK = 128
CH = (EW + CHUNK - 1) // CHUNK
EWP = CH * CHUNK
NPAD = 10112
RPT = NPAD // NS
DEGW = 16


def _sc_mesh():
    return plsc.VectorSubcoreMesh(core_axis_name="c", subcore_axis_name="s")


_SC_PARAMS = pltpu.CompilerParams(use_tc_tiling_on_sc=False)


@functools.partial(
    pl.kernel,
    out_type=jax.ShapeDtypeStruct((NC, NPAD, DEGW), jnp.float32),
    mesh=_sc_mesh(),
    compiler_params=_SC_PARAMS,
    scratch_types=[
        pltpu.VMEM((CH, CHUNK), jnp.int32),
        pltpu.VMEM((CHUNK, DEGW), jnp.float32),
        pltpu.VMEM_SHARED((NPAD, DEGW), jnp.float32),
    ],
)
def _sc_degree(dstp_hbm, ones_hbm, zeros_hbm, out_hbm, dst_v, ones_v, acc):
    c = lax.axis_index("c")
    s = lax.axis_index("s")
    w = c * NS + s
    pltpu.sync_copy(zeros_hbm, acc.at[pl.ds(s * RPT, RPT)])
    pltpu.sync_copy(ones_hbm, ones_v)
    pltpu.sync_copy(dstp_hbm.at[w], dst_v)
    plsc.subcore_barrier()

    def body(j, carry):
        pltpu.sync_copy(ones_v, acc.at[dst_v.at[j]], add=True)
        return carry

    lax.fori_loop(0, CH, body, 0)
    plsc.subcore_barrier()
    pltpu.sync_copy(acc.at[pl.ds(s * RPT, RPT)],
                    out_hbm.at[c, pl.ds(s * RPT, RPT)])


def _make_sc_scatter(W):
    @functools.partial(
        pl.kernel,
        out_type=jax.ShapeDtypeStruct((NC, NPAD, W), jnp.float32),
        mesh=_sc_mesh(),
        compiler_params=_SC_PARAMS,
        scratch_types=[
            pltpu.VMEM((CH, CHUNK), jnp.int32),
            pltpu.VMEM((CH, CHUNK), jnp.int32),
            pltpu.VMEM((CHUNK, W), jnp.float32),
            pltpu.VMEM_SHARED((NPAD, W), jnp.float32),
            pltpu.SemaphoreType.DMA,
        ],
    )
    def k(hs_hbm, srcp_hbm, dstp_hbm, zeros_hbm, out_hbm,
          src_v, dst_v, rows_v, acc, sem):
        c = lax.axis_index("c")
        s = lax.axis_index("s")
        w = c * NS + s
        pltpu.sync_copy(zeros_hbm, acc.at[pl.ds(s * RPT, RPT)])
        pltpu.sync_copy(srcp_hbm.at[w], src_v)
        pltpu.sync_copy(dstp_hbm.at[w], dst_v)
        plsc.subcore_barrier()

        def body(j, carry):
            pltpu.async_copy(hs_hbm.at[src_v.at[j]], rows_v, sem).wait()
            pltpu.sync_copy(rows_v, acc.at[dst_v.at[j]], add=True)
            return carry

        lax.fori_loop(0, CH, body, 0)
        plsc.subcore_barrier()
        pltpu.sync_copy(acc.at[pl.ds(s * RPT, RPT)],
                        out_hbm.at[c, pl.ds(s * RPT, RPT)])

    return k


_sc_scatter32 = _make_sc_scatter(32)
_sc_scatter64 = _make_sc_scatter(64)


def _tc1(degp_ref, x_ref, w1_ref, dinv_ref, h1s_ref):
    d = degp_ref[0] + degp_ref[1] + 1.0
    dinv = lax.rsqrt(d)
    dinv_ref[...] = dinv
    h = jnp.dot(x_ref[...], w1_ref[...], preferred_element_type=jnp.float32)
    h1s_ref[...] = dinv * h


def _tc_stage(pp_ref, hs_ref, dinv_ref, b_ref, wn_ref, x_ref, hsn_ref):
    dinv = dinv_ref[...]
    xk = jnp.maximum(dinv * (pp_ref[0] + pp_ref[1] + hs_ref[...]) + b_ref[...],
                     0.0)
    x_ref[...] = xk
    hsn_ref[...] = dinv * jnp.dot(xk, wn_ref[...],
                                  preferred_element_type=jnp.float32)


def _tc4(pp_ref, h3s_ref, dinv_ref, b3_ref, x1_ref, x2_ref,
         wa_ref, wb_ref, wc_ref, bl1_ref, xls_ref):
    dinv = dinv_ref[...]
    x3 = jnp.maximum(
        dinv * (pp_ref[0] + pp_ref[1] + h3s_ref[...]) + b3_ref[...], 0.0)
    xl = (jnp.dot(x1_ref[...], wa_ref[...], preferred_element_type=jnp.float32)
          + jnp.dot(x2_ref[...], wb_ref[...], preferred_element_type=jnp.float32)
          + jnp.dot(x3, wc_ref[...], preferred_element_type=jnp.float32)
          + bl1_ref[...])
    xls_ref[...] = dinv * xl


def _mm(a, b):
    return jnp.dot(a, b, preferred_element_type=jnp.float32)


def _attend(Qp, K, V):
    outs = []
    for h in range(NUM_HEADS):
        sl = slice(h * HD, (h + 1) * HD)
        Qh = Qp[:, sl]
        Kh = K[:, sl]
        Vh = V[:, sl]
        logits = lax.dot_general(Qh, Kh, (((1,), (1,)), ((), ()))) * SCALE
        m = jnp.max(logits, axis=-1, keepdims=True)
        e = jnp.exp(logits - m)
        a = e / jnp.sum(e, axis=-1, keepdims=True)
        outs.append(Qh + _mm(a, Vh))
    return jnp.concatenate(outs, axis=1)


def _tc_attn(p40_ref, p41_ref, xls_ref, dinv_ref,
             wk1_ref, bk1_ref, wv1_ref, bv1_ref,
             s1_ref, wq1_ref, bq1_ref, wo1_ref, bo1_ref,
             wq2_ref, bq2_ref, wk2_ref, bk2_ref, wv2_ref, bv2_ref,
             wo2_ref, bo2_ref,
             s2_ref, wq3_ref, bq3_ref, wk3_ref, bk3_ref, wv3_ref, bv3_ref,
             wo3_ref, bo3_ref,
             wg_ref, bg_ref, wl1_ref, bl1_ref, wl2_ref, bl2_ref,
             out_ref):
    dinv = dinv_ref[...]
    pxl = dinv * (p40_ref[...] + p41_ref[...] + xls_ref[...])
    K = _mm(pxl, wk1_ref[...]) + bk1_ref[...]
    V = _mm(pxl, wv1_ref[...]) + bv1_ref[...]

    Qp1 = _mm(s1_ref[...], wq1_ref[...]) + bq1_ref[...]
    O = _attend(Qp1, K, V)
    X = O + jnp.maximum(_mm(O, wo1_ref[...]) + bo1_ref[...], 0.0)

    Qp2 = _mm(X, wq2_ref[...]) + bq2_ref[...]
    K2 = _mm(X, wk2_ref[...]) + bk2_ref[...]
    V2 = _mm(X, wv2_ref[...]) + bv2_ref[...]
    O2 = _attend(Qp2, K2, V2)
    X2 = O2 + jnp.maximum(_mm(O2, wo2_ref[...]) + bo2_ref[...], 0.0)

    Qp3 = _mm(s2_ref[...], wq3_ref[...]) + bq3_ref[...]
    K3 = _mm(X2, wk3_ref[...]) + bk3_ref[...]
    V3 = _mm(X2, wv3_ref[...]) + bv3_ref[...]
    O3 = _attend(Qp3, K3, V3)
    X3 = O3 + jnp.maximum(_mm(O3, wo3_ref[...]) + bo3_ref[...], 0.0)

    g = _mm(X3, wg_ref[...]) + bg_ref[...]
    h = jnp.maximum(_mm(g, wl1_ref[...]) + bl1_ref[...], 0.0)
    o = _mm(h, wl2_ref[...]) + bl2_ref[...]
    m = jnp.max(o, axis=-1, keepdims=True)
    lse = jnp.log(jnp.sum(jnp.exp(o - m), axis=-1, keepdims=True)) + m
    out_ref[...] = (o - lse)[None]


def kernel(x, edge_index, batch, params):
    p = params
    f32 = jnp.float32

    src = edge_index[0].reshape(NW, EW)
    dst = edge_index[1].reshape(NW, EW)
    pad_s = jnp.zeros((NW, EWP - EW), jnp.int32)
    pad_d = jnp.full((NW, EWP - EW), N, jnp.int32)
    srcp = jnp.concatenate([src, pad_s], axis=1).reshape(NW, CH, CHUNK)
    dstp = jnp.concatenate([dst, pad_d], axis=1).reshape(NW, CH, CHUNK)

    ones_deg = jnp.ones((CHUNK, DEGW), f32)
    zeros_deg = jnp.zeros((RPT, DEGW), f32)
    zeros32 = jnp.zeros((RPT, 32), f32)
    zeros64 = jnp.zeros((RPT, 64), f32)

    degp = _sc_degree(dstp, ones_deg, zeros_deg)

    dinv, h1s = pl.pallas_call(
        _tc1,
        out_shape=[jax.ShapeDtypeStruct((N, 1), f32),
                   jax.ShapeDtypeStruct((N, 32), f32)],
    )(degp[:, :N, 0:1], x, p["conv1"]["W"])

    p1 = _sc_scatter32(h1s, srcp, dstp, zeros32)
    x1, h2s = pl.pallas_call(
        _tc_stage,
        out_shape=[jax.ShapeDtypeStruct((N, 32), f32),
                   jax.ShapeDtypeStruct((N, 32), f32)],
    )(p1[:, :N], h1s, dinv, p["conv1"]["b"].reshape(1, 32), p["conv2"]["W"])

    p2 = _sc_scatter32(h2s, srcp, dstp, zeros32)
    x2, h3s = pl.pallas_call(
        _tc_stage,
        out_shape=[jax.ShapeDtypeStruct((N, 32), f32),
                   jax.ShapeDtypeStruct((N, 32), f32)],
    )(p2[:, :N], h2s, dinv, p["conv2"]["b"].reshape(1, 32), p["conv3"]["W"])

    p3 = _sc_scatter32(h3s, srcp, dstp, zeros32)
    wl1 = p["gmt_lin1"]["W"]
    xls = pl.pallas_call(
        _tc4,
        out_shape=jax.ShapeDtypeStruct((N, HID), f32),
    )(p3[:, :N], h3s, dinv, p["conv3"]["b"].reshape(1, 32), x1, x2,
      wl1[:32], wl1[32:64], wl1[64:], p["gmt_lin1"]["b"].reshape(1, HID))

    p4 = _sc_scatter64(xls, srcp, dstp, zeros64)

    gblk = lambda W: pl.BlockSpec((NPG, W), lambda g: (g, 0))
    full = lambda shp: pl.BlockSpec(shp, lambda g: (0,) * len(shp))

    def lin_specs(*names):
        specs = []
        args = []
        for name in names:
            W = p[name]["W"]
            b = p[name]["b"]
            specs += [full(W.shape), full((1, b.shape[0]))]
            args += [W, b.reshape(1, -1)]
        return specs, args

    in_specs = [gblk(HID), gblk(HID), gblk(HID), gblk(1)]
    s1 = p["S1"].reshape(NUM_SEEDS, HID)
    s2 = p["S2"].reshape(1, HID)

    wspecs, wargs = lin_specs("mab1_layer_k", "mab1_layer_v")
    in_specs += wspecs
    in_specs += [full((NUM_SEEDS, HID))]
    qspecs, qargs = lin_specs("mab1_fc_q")
    in_specs += qspecs
    ospecs, oargs = lin_specs("mab1_fc_o")
    in_specs += ospecs
    m2specs, m2args = lin_specs("mab2_fc_q", "mab2_layer_k", "mab2_layer_v",
                                "mab2_fc_o")
    in_specs += m2specs
    in_specs += [full((1, HID))]
    m3specs, m3args = lin_specs("mab3_fc_q", "mab3_layer_k", "mab3_layer_v",
                                "mab3_fc_o")
    in_specs += m3specs
    tspecs, targs = lin_specs("gmt_lin2", "lin1", "lin2")
    in_specs += tspecs

    out = pl.pallas_call(
        _tc_attn,
        grid=(NUM_GRAPHS,),
        in_specs=in_specs,
        out_specs=pl.BlockSpec((1, 1, 10), lambda g: (g, 0, 0)),
        out_shape=jax.ShapeDtypeStruct((NUM_GRAPHS, 1, 10), f32),
    )(p4[0, :N], p4[1, :N], xls, dinv,
      *wargs, s1, *qargs, *oargs, *m2args, s2, *m3args, *targs)

    return out.reshape(NUM_GRAPHS, 10)


# speedup vs baseline: 17.0284x; 1.0212x over previous
import functools
import math

import jax
import jax.numpy as jnp
from jax import lax
from jax.experimental import pallas as pl
from jax.experimental.pallas import tpu as pltpu
from jax.experimental.pallas import tpu_sc as plsc

N = 10000
E = 320000
NUM_GRAPHS = 50
NPG = 200
HID = 64
NUM_HEADS = 4
HD = HID // NUM_HEADS
NUM_SEEDS = 50
SCALE = 1.0 / math.sqrt(HID)

NC = 2
NS = 16
NW = NC * NS
EW = E // NW
CHUNK = 128
NBUF = 4
CH = 80
EWP = CH * CHUNK
NPAD = 10112
RPT = NPAD // NS
DEGW = 16


def _sc_mesh():
    return plsc.VectorSubcoreMesh(core_axis_name="c", subcore_axis_name="s")


_SC_PARAMS = pltpu.CompilerParams(use_tc_tiling_on_sc=False)


@functools.partial(
    pl.kernel,
    out_type=jax.ShapeDtypeStruct((NC, NPAD, DEGW), jnp.float32),
    mesh=_sc_mesh(),
    compiler_params=_SC_PARAMS,
    scratch_types=[
        pltpu.VMEM((CH, CHUNK), jnp.int32),
        pltpu.VMEM((CHUNK, DEGW), jnp.float32),
        pltpu.VMEM_SHARED((NPAD, DEGW), jnp.float32),
    ],
)
def _sc_degree(dstp_hbm, ones_hbm, zeros_hbm, out_hbm, dst_v, ones_v, acc):
    c = lax.axis_index("c")
    s = lax.axis_index("s")
    w = c * NS + s
    pltpu.sync_copy(zeros_hbm, acc.at[pl.ds(s * RPT, RPT)])
    pltpu.sync_copy(ones_hbm, ones_v)
    pltpu.sync_copy(dstp_hbm.at[w], dst_v)
    plsc.subcore_barrier()

    def body(j, carry):
        pltpu.sync_copy(ones_v, acc.at[dst_v.at[j]], add=True)
        return carry

    lax.fori_loop(0, CH, body, 0)
    plsc.subcore_barrier()
    pltpu.sync_copy(acc.at[pl.ds(s * RPT, RPT)],
                    out_hbm.at[c, pl.ds(s * RPT, RPT)])


def _make_sc_scatter(W):
    @functools.partial(
        pl.kernel,
        out_type=jax.ShapeDtypeStruct((NC, NPAD, W), jnp.float32),
        mesh=_sc_mesh(),
        compiler_params=_SC_PARAMS,
        scratch_types=[
            pltpu.VMEM((CH, CHUNK), jnp.int32),
            pltpu.VMEM((CH, CHUNK), jnp.int32),
            [pltpu.VMEM((CHUNK, W), jnp.float32) for _ in range(NBUF)],
            pltpu.VMEM_SHARED((NPAD, W), jnp.float32),
            [pltpu.SemaphoreType.DMA for _ in range(NBUF)],
        ],
    )
    def k(hs_hbm, srcp_hbm, dstp_hbm, zeros_hbm, out_hbm,
          src_v, dst_v, rows, acc, sems):
        c = lax.axis_index("c")
        s = lax.axis_index("s")
        w = c * NS + s
        pltpu.sync_copy(zeros_hbm, acc.at[pl.ds(s * RPT, RPT)])
        pltpu.sync_copy(srcp_hbm.at[w], src_v)
        pltpu.sync_copy(dstp_hbm.at[w], dst_v)
        for b in range(NBUF):
            pltpu.async_copy(hs_hbm.at[src_v.at[b]], rows[b], sems[b])
        plsc.subcore_barrier()

        def body(i, carry):
            for b in range(NBUF):
                j = i * NBUF + b
                pltpu.make_async_copy(hs_hbm.at[src_v.at[j]],
                                      rows[b], sems[b]).wait()
                pltpu.sync_copy(rows[b], acc.at[dst_v.at[j]], add=True)

                @pl.when(j + NBUF < CH)
                def _():
                    pltpu.async_copy(hs_hbm.at[src_v.at[j + NBUF]],
                                     rows[b], sems[b])
            return carry

        lax.fori_loop(0, CH // NBUF, body, 0)
        plsc.subcore_barrier()
        pltpu.sync_copy(acc.at[pl.ds(s * RPT, RPT)],
                        out_hbm.at[c, pl.ds(s * RPT, RPT)])

    return k


_sc_scatter32 = _make_sc_scatter(32)
_sc_scatter64 = _make_sc_scatter(64)


def _tc1(degp_ref, x_ref, w1_ref, dinv_ref, h1s_ref):
    d = degp_ref[0] + degp_ref[1] + 1.0
    dinv = lax.rsqrt(d)
    dinv_ref[...] = dinv
    h = jnp.dot(x_ref[...], w1_ref[...], preferred_element_type=jnp.float32)
    h1s_ref[...] = dinv * h


def _tc_stage(pp_ref, hs_ref, dinv_ref, b_ref, wn_ref, x_ref, hsn_ref):
    dinv = dinv_ref[...]
    xk = jnp.maximum(dinv * (pp_ref[0] + pp_ref[1] + hs_ref[...]) + b_ref[...],
                     0.0)
    x_ref[...] = xk
    hsn_ref[...] = dinv * jnp.dot(xk, wn_ref[...],
                                  preferred_element_type=jnp.float32)


def _tc4(pp_ref, h3s_ref, dinv_ref, b3_ref, x1_ref, x2_ref,
         wa_ref, wb_ref, wc_ref, bl1_ref, xls_ref):
    dinv = dinv_ref[...]
    x3 = jnp.maximum(
        dinv * (pp_ref[0] + pp_ref[1] + h3s_ref[...]) + b3_ref[...], 0.0)
    xl = (jnp.dot(x1_ref[...], wa_ref[...], preferred_element_type=jnp.float32)
          + jnp.dot(x2_ref[...], wb_ref[...], preferred_element_type=jnp.float32)
          + jnp.dot(x3, wc_ref[...], preferred_element_type=jnp.float32)
          + bl1_ref[...])
    xls_ref[...] = dinv * xl


def _mm(a, b):
    return jnp.dot(a, b, preferred_element_type=jnp.float32)


def _attend(Qp, K, V):
    outs = []
    for h in range(NUM_HEADS):
        sl = slice(h * HD, (h + 1) * HD)
        Qh = Qp[:, sl]
        Kh = K[:, sl]
        Vh = V[:, sl]
        logits = lax.dot_general(Qh, Kh, (((1,), (1,)), ((), ()))) * SCALE
        m = jnp.max(logits, axis=-1, keepdims=True)
        e = jnp.exp(logits - m)
        a = e / jnp.sum(e, axis=-1, keepdims=True)
        outs.append(Qh + _mm(a, Vh))
    return jnp.concatenate(outs, axis=1)


def _tc_attn(p40_ref, p41_ref, xls_ref, dinv_ref,
             wk1_ref, bk1_ref, wv1_ref, bv1_ref,
             s1_ref, wq1_ref, bq1_ref, wo1_ref, bo1_ref,
             wq2_ref, bq2_ref, wk2_ref, bk2_ref, wv2_ref, bv2_ref,
             wo2_ref, bo2_ref,
             s2_ref, wq3_ref, bq3_ref, wk3_ref, bk3_ref, wv3_ref, bv3_ref,
             wo3_ref, bo3_ref,
             wg_ref, bg_ref, wl1_ref, bl1_ref, wl2_ref, bl2_ref,
             out_ref):
    dinv = dinv_ref[...]
    pxl = dinv * (p40_ref[...] + p41_ref[...] + xls_ref[...])
    K = _mm(pxl, wk1_ref[...]) + bk1_ref[...]
    V = _mm(pxl, wv1_ref[...]) + bv1_ref[...]

    Qp1 = _mm(s1_ref[...], wq1_ref[...]) + bq1_ref[...]
    O = _attend(Qp1, K, V)
    X = O + jnp.maximum(_mm(O, wo1_ref[...]) + bo1_ref[...], 0.0)

    Qp2 = _mm(X, wq2_ref[...]) + bq2_ref[...]
    K2 = _mm(X, wk2_ref[...]) + bk2_ref[...]
    V2 = _mm(X, wv2_ref[...]) + bv2_ref[...]
    O2 = _attend(Qp2, K2, V2)
    X2 = O2 + jnp.maximum(_mm(O2, wo2_ref[...]) + bo2_ref[...], 0.0)

    Qp3 = _mm(s2_ref[...], wq3_ref[...]) + bq3_ref[...]
    K3 = _mm(X2, wk3_ref[...]) + bk3_ref[...]
    V3 = _mm(X2, wv3_ref[...]) + bv3_ref[...]
    O3 = _attend(Qp3, K3, V3)
    X3 = O3 + jnp.maximum(_mm(O3, wo3_ref[...]) + bo3_ref[...], 0.0)

    g = _mm(X3, wg_ref[...]) + bg_ref[...]
    h = jnp.maximum(_mm(g, wl1_ref[...]) + bl1_ref[...], 0.0)
    o = _mm(h, wl2_ref[...]) + bl2_ref[...]
    m = jnp.max(o, axis=-1, keepdims=True)
    lse = jnp.log(jnp.sum(jnp.exp(o - m), axis=-1, keepdims=True)) + m
    out_ref[...] = (o - lse)[None]


def kernel(x, edge_index, batch, params):
    p = params
    f32 = jnp.float32

    src = edge_index[0].reshape(NW, EW)
    dst = edge_index[1].reshape(NW, EW)
    pad_s = jnp.zeros((NW, EWP - EW), jnp.int32)
    pad_d = jnp.full((NW, EWP - EW), N, jnp.int32)
    srcp = jnp.concatenate([src, pad_s], axis=1).reshape(NW, CH, CHUNK)
    dstp = jnp.concatenate([dst, pad_d], axis=1).reshape(NW, CH, CHUNK)

    ones_deg = jnp.ones((CHUNK, DEGW), f32)
    zeros_deg = jnp.zeros((RPT, DEGW), f32)
    zeros32 = jnp.zeros((RPT, 32), f32)
    zeros64 = jnp.zeros((RPT, 64), f32)

    degp = _sc_degree(dstp, ones_deg, zeros_deg)

    dinv, h1s = pl.pallas_call(
        _tc1,
        out_shape=[jax.ShapeDtypeStruct((N, 1), f32),
                   jax.ShapeDtypeStruct((N, 32), f32)],
    )(degp[:, :N, 0:1], x, p["conv1"]["W"])

    p1 = _sc_scatter32(h1s, srcp, dstp, zeros32)
    x1, h2s = pl.pallas_call(
        _tc_stage,
        out_shape=[jax.ShapeDtypeStruct((N, 32), f32),
                   jax.ShapeDtypeStruct((N, 32), f32)],
    )(p1[:, :N], h1s, dinv, p["conv1"]["b"].reshape(1, 32), p["conv2"]["W"])

    p2 = _sc_scatter32(h2s, srcp, dstp, zeros32)
    x2, h3s = pl.pallas_call(
        _tc_stage,
        out_shape=[jax.ShapeDtypeStruct((N, 32), f32),
                   jax.ShapeDtypeStruct((N, 32), f32)],
    )(p2[:, :N], h2s, dinv, p["conv2"]["b"].reshape(1, 32), p["conv3"]["W"])

    p3 = _sc_scatter32(h3s, srcp, dstp, zeros32)
    wl1 = p["gmt_lin1"]["W"]
    xls = pl.pallas_call(
        _tc4,
        out_shape=jax.ShapeDtypeStruct((N, HID), f32),
    )(p3[:, :N], h3s, dinv, p["conv3"]["b"].reshape(1, 32), x1, x2,
      wl1[:32], wl1[32:64], wl1[64:], p["gmt_lin1"]["b"].reshape(1, HID))

    p4 = _sc_scatter64(xls, srcp, dstp, zeros64)

    gblk = lambda W: pl.BlockSpec((NPG, W), lambda g: (g, 0))
    full = lambda shp: pl.BlockSpec(shp, lambda g: (0,) * len(shp))

    def lin_specs(*names):
        specs = []
        args = []
        for name in names:
            W = p[name]["W"]
            b = p[name]["b"]
            specs += [full(W.shape), full((1, b.shape[0]))]
            args += [W, b.reshape(1, -1)]
        return specs, args

    in_specs = [gblk(HID), gblk(HID), gblk(HID), gblk(1)]
    s1 = p["S1"].reshape(NUM_SEEDS, HID)
    s2 = p["S2"].reshape(1, HID)

    wspecs, wargs = lin_specs("mab1_layer_k", "mab1_layer_v")
    in_specs += wspecs
    in_specs += [full((NUM_SEEDS, HID))]
    qspecs, qargs = lin_specs("mab1_fc_q")
    in_specs += qspecs
    ospecs, oargs = lin_specs("mab1_fc_o")
    in_specs += ospecs
    m2specs, m2args = lin_specs("mab2_fc_q", "mab2_layer_k", "mab2_layer_v",
                                "mab2_fc_o")
    in_specs += m2specs
    in_specs += [full((1, HID))]
    m3specs, m3args = lin_specs("mab3_fc_q", "mab3_layer_k", "mab3_layer_v",
                                "mab3_fc_o")
    in_specs += m3specs
    tspecs, targs = lin_specs("gmt_lin2", "lin1", "lin2")
    in_specs += tspecs

    out = pl.pallas_call(
        _tc_attn,
        grid=(NUM_GRAPHS,),
        in_specs=in_specs,
        out_specs=pl.BlockSpec((1, 1, 10), lambda g: (g, 0, 0)),
        out_shape=jax.ShapeDtypeStruct((NUM_GRAPHS, 1, 10), f32),
    )(p4[0, :N], p4[1, :N], xls, dinv,
      *wargs, s1, *qargs, *oargs, *m2args, s2, *m3args, *targs)

    return out.reshape(NUM_GRAPHS, 10)


# speedup vs baseline: 21.2579x; 1.2484x over previous
import functools
import math

import jax
import jax.numpy as jnp
from jax import lax
from jax.experimental import pallas as pl
from jax.experimental.pallas import tpu as pltpu
from jax.experimental.pallas import tpu_sc as plsc

N = 10000
E = 320000
NUM_GRAPHS = 50
NPG = 200
HID = 64
NUM_HEADS = 4
HD = HID // NUM_HEADS
NUM_SEEDS = 50
SCALE = 1.0 / math.sqrt(HID)

NC = 2
NS = 16
NW = NC * NS
EW = E // NW
CHUNK = 128
NBUF = 4
CH = 80
EWP = CH * CHUNK
NPAD = 10112
RPT = NPAD // NS
DEGW = 16


def _sc_mesh():
    return plsc.VectorSubcoreMesh(core_axis_name="c", subcore_axis_name="s")


_SC_PARAMS = pltpu.CompilerParams(use_tc_tiling_on_sc=False)


@functools.partial(
    pl.kernel,
    out_type=jax.ShapeDtypeStruct((NC, NPAD, DEGW), jnp.float32),
    mesh=_sc_mesh(),
    compiler_params=_SC_PARAMS,
    scratch_types=[
        pltpu.VMEM((CH, CHUNK), jnp.int32),
        pltpu.VMEM((CHUNK, DEGW), jnp.float32),
        pltpu.VMEM_SHARED((NPAD, DEGW), jnp.float32),
    ],
)
def _sc_degree(dstp_hbm, ones_hbm, zeros_hbm, out_hbm, dst_v, ones_v, acc):
    c = lax.axis_index("c")
    s = lax.axis_index("s")
    w = c * NS + s
    pltpu.sync_copy(zeros_hbm, acc.at[pl.ds(s * RPT, RPT)])
    pltpu.sync_copy(ones_hbm, ones_v)
    pltpu.sync_copy(dstp_hbm.at[w], dst_v)
    plsc.subcore_barrier()

    def body(j, carry):
        pltpu.sync_copy(ones_v, acc.at[dst_v.at[j]], add=True)
        return carry

    lax.fori_loop(0, CH, body, 0)
    plsc.subcore_barrier()
    pltpu.sync_copy(acc.at[pl.ds(s * RPT, RPT)],
                    out_hbm.at[c, pl.ds(s * RPT, RPT)])


def _make_sc_scatter(W):
    @functools.partial(
        pl.kernel,
        out_type=jax.ShapeDtypeStruct((NC, NPAD, W), jnp.float32),
        mesh=_sc_mesh(),
        compiler_params=_SC_PARAMS,
        scratch_types=[
            pltpu.VMEM((CH, CHUNK), jnp.int32),
            pltpu.VMEM((CH, CHUNK), jnp.int32),
            [pltpu.VMEM((CHUNK, W), jnp.float32) for _ in range(NBUF)],
            pltpu.VMEM_SHARED((NPAD, W), jnp.float32),
            [pltpu.SemaphoreType.DMA for _ in range(NBUF)],
        ],
    )
    def k(hs_hbm, srcp_hbm, dstp_hbm, zeros_hbm, out_hbm,
          src_v, dst_v, rows, acc, sems):
        c = lax.axis_index("c")
        s = lax.axis_index("s")
        w = c * NS + s
        pltpu.sync_copy(zeros_hbm, acc.at[pl.ds(s * RPT, RPT)])
        pltpu.sync_copy(srcp_hbm.at[w], src_v)
        pltpu.sync_copy(dstp_hbm.at[w], dst_v)
        for b in range(NBUF):
            pltpu.async_copy(hs_hbm.at[src_v.at[b]], rows[b], sems[b])
        plsc.subcore_barrier()

        def body(i, carry):
            for b in range(NBUF):
                j = i * NBUF + b
                pltpu.make_async_copy(hs_hbm.at[src_v.at[j]],
                                      rows[b], sems[b]).wait()
                pltpu.sync_copy(rows[b], acc.at[dst_v.at[j]], add=True)

                @pl.when(j + NBUF < CH)
                def _():
                    pltpu.async_copy(hs_hbm.at[src_v.at[j + NBUF]],
                                     rows[b], sems[b])
            return carry

        lax.fori_loop(0, CH // NBUF, body, 0)
        plsc.subcore_barrier()
        pltpu.sync_copy(acc.at[pl.ds(s * RPT, RPT)],
                        out_hbm.at[c, pl.ds(s * RPT, RPT)])

    return k


_sc_scatter32 = _make_sc_scatter(32)
_sc_scatter64 = _make_sc_scatter(64)


def _tc1(degp_ref, x_ref, w1_ref, dinv_ref, h1s_ref):
    d = degp_ref[0] + degp_ref[1] + 1.0
    dinv = lax.rsqrt(d)
    dinv_ref[...] = dinv
    h = jnp.dot(x_ref[...], w1_ref[...], preferred_element_type=jnp.float32)
    h1s_ref[...] = dinv * h


def _tc_stage(pp_ref, hs_ref, dinv_ref, b_ref, wn_ref, x_ref, hsn_ref):
    dinv = dinv_ref[...]
    xk = jnp.maximum(dinv * (pp_ref[0] + pp_ref[1] + hs_ref[...]) + b_ref[...],
                     0.0)
    x_ref[...] = xk
    hsn_ref[...] = dinv * jnp.dot(xk, wn_ref[...],
                                  preferred_element_type=jnp.float32)


def _tc4(pp_ref, h3s_ref, dinv_ref, b3_ref, x1_ref, x2_ref,
         wa_ref, wb_ref, wc_ref, bl1_ref, xls_ref):
    dinv = dinv_ref[...]
    x3 = jnp.maximum(
        dinv * (pp_ref[0] + pp_ref[1] + h3s_ref[...]) + b3_ref[...], 0.0)
    xl = (jnp.dot(x1_ref[...], wa_ref[...], preferred_element_type=jnp.float32)
          + jnp.dot(x2_ref[...], wb_ref[...], preferred_element_type=jnp.float32)
          + jnp.dot(x3, wc_ref[...], preferred_element_type=jnp.float32)
          + bl1_ref[...])
    xls_ref[...] = dinv * xl


def _mm(a, b):
    return jnp.dot(a, b, preferred_element_type=jnp.float32)


def _head_masks():
    col = lax.broadcasted_iota(jnp.int32, (1, HID), 1)
    return [(col // HD == h).astype(jnp.float32) for h in range(NUM_HEADS)]


def _stack_heads(Qp, masks):
    return jnp.concatenate([Qp * m for m in masks], axis=0)


def _merge_heads(AV, nq):
    return jnp.concatenate(
        [AV[h * nq:(h + 1) * nq, h * HD:(h + 1) * HD]
         for h in range(NUM_HEADS)], axis=1)


def _attend_stacked(Qstack, Qp, K, V, nq):
    logits = lax.dot_general(Qstack, K, (((1,), (1,)), ((), ()))) * SCALE
    m = jnp.max(logits, axis=-1, keepdims=True)
    e = jnp.exp(logits - m)
    a = e / jnp.sum(e, axis=-1, keepdims=True)
    return Qp + _merge_heads(_mm(a, V), nq)


def _tc_attn(p40_ref, p41_ref, xls_ref, dinv_ref,
             wk1_ref, bk1_ref, wv1_ref, bv1_ref,
             s1_ref, wq1_ref, bq1_ref, wo1_ref, bo1_ref,
             wq2_ref, bq2_ref, wk2_ref, bk2_ref, wv2_ref, bv2_ref,
             wo2_ref, bo2_ref,
             s2_ref, wq3_ref, bq3_ref, wk3_ref, bk3_ref, wv3_ref, bv3_ref,
             wo3_ref, bo3_ref,
             wg_ref, bg_ref, wl1_ref, bl1_ref, wl2_ref, bl2_ref,
             out_ref):
    masks = _head_masks()
    dinv = dinv_ref[...]
    pxl = dinv * (p40_ref[...] + p41_ref[...] + xls_ref[...])
    K1 = _mm(pxl, wk1_ref[...]) + bk1_ref[...]
    V1 = _mm(pxl, wv1_ref[...]) + bv1_ref[...]

    Qp1 = _mm(s1_ref[...], wq1_ref[...]) + bq1_ref[...]
    Qs1 = _stack_heads(Qp1, masks)
    O1 = jnp.concatenate(
        [_attend_stacked(Qs1, Qp1, K1[g * NPG:(g + 1) * NPG],
                         V1[g * NPG:(g + 1) * NPG], NUM_SEEDS)
         for g in range(NUM_GRAPHS)], axis=0)
    X1 = O1 + jnp.maximum(_mm(O1, wo1_ref[...]) + bo1_ref[...], 0.0)

    Qp2 = _mm(X1, wq2_ref[...]) + bq2_ref[...]
    K2 = _mm(X1, wk2_ref[...]) + bk2_ref[...]
    V2 = _mm(X1, wv2_ref[...]) + bv2_ref[...]
    NS2 = NUM_SEEDS
    O2 = jnp.concatenate(
        [_attend_stacked(_stack_heads(Qp2[g * NS2:(g + 1) * NS2], masks),
                         Qp2[g * NS2:(g + 1) * NS2],
                         K2[g * NS2:(g + 1) * NS2],
                         V2[g * NS2:(g + 1) * NS2], NS2)
         for g in range(NUM_GRAPHS)], axis=0)
    X2 = O2 + jnp.maximum(_mm(O2, wo2_ref[...]) + bo2_ref[...], 0.0)

    Qp3 = _mm(s2_ref[...], wq3_ref[...]) + bq3_ref[...]
    Qs3 = _stack_heads(Qp3, masks)
    K3 = _mm(X2, wk3_ref[...]) + bk3_ref[...]
    V3 = _mm(X2, wv3_ref[...]) + bv3_ref[...]
    O3 = jnp.concatenate(
        [_attend_stacked(Qs3, Qp3, K3[g * NS2:(g + 1) * NS2],
                         V3[g * NS2:(g + 1) * NS2], 1)
         for g in range(NUM_GRAPHS)], axis=0)
    X3 = O3 + jnp.maximum(_mm(O3, wo3_ref[...]) + bo3_ref[...], 0.0)

    g = _mm(X3, wg_ref[...]) + bg_ref[...]
    h = jnp.maximum(_mm(g, wl1_ref[...]) + bl1_ref[...], 0.0)
    o = _mm(h, wl2_ref[...]) + bl2_ref[...]
    m = jnp.max(o, axis=-1, keepdims=True)
    lse = jnp.log(jnp.sum(jnp.exp(o - m), axis=-1, keepdims=True)) + m
    out_ref[...] = o - lse


def kernel(x, edge_index, batch, params):
    p = params
    f32 = jnp.float32

    src = edge_index[0].reshape(NW, EW)
    dst = edge_index[1].reshape(NW, EW)
    pad_s = jnp.zeros((NW, EWP - EW), jnp.int32)
    pad_d = jnp.full((NW, EWP - EW), N, jnp.int32)
    srcp = jnp.concatenate([src, pad_s], axis=1).reshape(NW, CH, CHUNK)
    dstp = jnp.concatenate([dst, pad_d], axis=1).reshape(NW, CH, CHUNK)

    ones_deg = jnp.ones((CHUNK, DEGW), f32)
    zeros_deg = jnp.zeros((RPT, DEGW), f32)
    zeros32 = jnp.zeros((RPT, 32), f32)
    zeros64 = jnp.zeros((RPT, 64), f32)

    degp = _sc_degree(dstp, ones_deg, zeros_deg)

    dinv, h1s = pl.pallas_call(
        _tc1,
        out_shape=[jax.ShapeDtypeStruct((N, 1), f32),
                   jax.ShapeDtypeStruct((N, 32), f32)],
    )(degp[:, :N, 0:1], x, p["conv1"]["W"])

    p1 = _sc_scatter32(h1s, srcp, dstp, zeros32)
    x1, h2s = pl.pallas_call(
        _tc_stage,
        out_shape=[jax.ShapeDtypeStruct((N, 32), f32),
                   jax.ShapeDtypeStruct((N, 32), f32)],
    )(p1[:, :N], h1s, dinv, p["conv1"]["b"].reshape(1, 32), p["conv2"]["W"])

    p2 = _sc_scatter32(h2s, srcp, dstp, zeros32)
    x2, h3s = pl.pallas_call(
        _tc_stage,
        out_shape=[jax.ShapeDtypeStruct((N, 32), f32),
                   jax.ShapeDtypeStruct((N, 32), f32)],
    )(p2[:, :N], h2s, dinv, p["conv2"]["b"].reshape(1, 32), p["conv3"]["W"])

    p3 = _sc_scatter32(h3s, srcp, dstp, zeros32)
    wl1 = p["gmt_lin1"]["W"]
    xls = pl.pallas_call(
        _tc4,
        out_shape=jax.ShapeDtypeStruct((N, HID), f32),
    )(p3[:, :N], h3s, dinv, p["conv3"]["b"].reshape(1, 32), x1, x2,
      wl1[:32], wl1[32:64], wl1[64:], p["gmt_lin1"]["b"].reshape(1, HID))

    p4 = _sc_scatter64(xls, srcp, dstp, zeros64)

    def lin_args(*names):
        args = []
        for name in names:
            args += [p[name]["W"], p[name]["b"].reshape(1, -1)]
        return args

    s1 = p["S1"].reshape(NUM_SEEDS, HID)
    s2 = p["S2"].reshape(1, HID)

    out = pl.pallas_call(
        _tc_attn,
        out_shape=jax.ShapeDtypeStruct((NUM_GRAPHS, 10), f32),
    )(p4[0, :N], p4[1, :N], xls, dinv,
      *lin_args("mab1_layer_k", "mab1_layer_v"),
      s1, *lin_args("mab1_fc_q"), *lin_args("mab1_fc_o"),
      *lin_args("mab2_fc_q", "mab2_layer_k", "mab2_layer_v", "mab2_fc_o"),
      s2, *lin_args("mab3_fc_q", "mab3_layer_k", "mab3_layer_v", "mab3_fc_o"),
      *lin_args("gmt_lin2", "lin1", "lin2"))

    return out


# speedup vs baseline: 22.3248x; 1.0502x over previous
import functools
import math

import jax
import jax.numpy as jnp
from jax import lax
from jax.experimental import pallas as pl
from jax.experimental.pallas import tpu as pltpu
from jax.experimental.pallas import tpu_sc as plsc

N = 10000
E = 320000
NUM_GRAPHS = 50
NPG = 200
HID = 64
NUM_HEADS = 4
HD = HID // NUM_HEADS
NUM_SEEDS = 50
SCALE = 1.0 / math.sqrt(HID)

NC = 2
NS = 16
NW = NC * NS
EW = E // NW
CHUNK = 128
NBUF = 4
CH = 80
EWP = CH * CHUNK
NPAD = 10112
RPT = NPAD // NS
DEGW = 16


def _sc_mesh():
    return plsc.VectorSubcoreMesh(core_axis_name="c", subcore_axis_name="s")


_SC_PARAMS = pltpu.CompilerParams(use_tc_tiling_on_sc=False)


@functools.partial(
    pl.kernel,
    out_type=jax.ShapeDtypeStruct((NC, NPAD, DEGW), jnp.float32),
    mesh=_sc_mesh(),
    compiler_params=_SC_PARAMS,
    scratch_types=[
        pltpu.VMEM((CH, CHUNK), jnp.int32),
        pltpu.VMEM((CHUNK, DEGW), jnp.float32),
        pltpu.VMEM_SHARED((NPAD, DEGW), jnp.float32),
    ],
)
def _sc_degree(dstp_hbm, ones_hbm, zeros_hbm, out_hbm, dst_v, ones_v, acc):
    c = lax.axis_index("c")
    s = lax.axis_index("s")
    w = c * NS + s
    pltpu.sync_copy(zeros_hbm, acc.at[pl.ds(s * RPT, RPT)])
    pltpu.sync_copy(ones_hbm, ones_v)
    pltpu.sync_copy(dstp_hbm.at[w], dst_v)
    plsc.subcore_barrier()

    def body(j, carry):
        pltpu.sync_copy(ones_v, acc.at[dst_v.at[j]], add=True)
        return carry

    lax.fori_loop(0, CH, body, 0)
    plsc.subcore_barrier()
    pltpu.sync_copy(acc.at[pl.ds(s * RPT, RPT)],
                    out_hbm.at[c, pl.ds(s * RPT, RPT)])


def _make_sc_scatter(W):
    @functools.partial(
        pl.kernel,
        out_type=jax.ShapeDtypeStruct((NC, NPAD, W), jnp.float32),
        mesh=_sc_mesh(),
        compiler_params=_SC_PARAMS,
        scratch_types=[
            pltpu.VMEM((CH, CHUNK), jnp.int32),
            pltpu.VMEM((CH, CHUNK), jnp.int32),
            [pltpu.VMEM((CHUNK, W), jnp.float32) for _ in range(NBUF)],
            pltpu.VMEM_SHARED((NPAD, W), jnp.float32),
            [pltpu.SemaphoreType.DMA for _ in range(NBUF)],
        ],
    )
    def k(hs_hbm, srcp_hbm, dstp_hbm, zeros_hbm, out_hbm,
          src_v, dst_v, rows, acc, sems):
        c = lax.axis_index("c")
        s = lax.axis_index("s")
        w = c * NS + s
        pltpu.sync_copy(zeros_hbm, acc.at[pl.ds(s * RPT, RPT)])
        pltpu.sync_copy(srcp_hbm.at[w], src_v)
        pltpu.sync_copy(dstp_hbm.at[w], dst_v)
        for b in range(NBUF):
            pltpu.async_copy(hs_hbm.at[src_v.at[b]], rows[b], sems[b])
        plsc.subcore_barrier()

        def body(i, carry):
            for b in range(NBUF):
                j = i * NBUF + b
                pltpu.make_async_copy(hs_hbm.at[src_v.at[j]],
                                      rows[b], sems[b]).wait()
                pltpu.sync_copy(rows[b], acc.at[dst_v.at[j]], add=True)

                @pl.when(j + NBUF < CH)
                def _():
                    pltpu.async_copy(hs_hbm.at[src_v.at[j + NBUF]],
                                     rows[b], sems[b])
            return carry

        lax.fori_loop(0, CH // NBUF, body, 0)
        plsc.subcore_barrier()
        pltpu.sync_copy(acc.at[pl.ds(s * RPT, RPT)],
                        out_hbm.at[c, pl.ds(s * RPT, RPT)])

    return k


_sc_scatter32 = _make_sc_scatter(32)
_sc_scatter64 = _make_sc_scatter(64)


def _tc1(degp_ref, x_ref, w1_ref, dinv_ref, h1s_ref):
    d = degp_ref[0, :N, 0:1] + degp_ref[1, :N, 0:1] + 1.0
    dinv = lax.rsqrt(d)
    dinv_ref[...] = dinv
    h = jnp.dot(x_ref[...], w1_ref[...], preferred_element_type=jnp.float32)
    h1s_ref[...] = dinv * h


def _tc_stage(pp_ref, hs_ref, dinv_ref, b_ref, wn_ref, x_ref, hsn_ref):
    dinv = dinv_ref[...]
    xk = jnp.maximum(
        dinv * (pp_ref[0, :N] + pp_ref[1, :N] + hs_ref[...]) + b_ref[...], 0.0)
    x_ref[...] = xk
    hsn_ref[...] = dinv * jnp.dot(xk, wn_ref[...],
                                  preferred_element_type=jnp.float32)


def _tc4(pp_ref, h3s_ref, dinv_ref, b3_ref, x1_ref, x2_ref,
         wa_ref, wb_ref, wc_ref, bl1_ref, xls_ref):
    dinv = dinv_ref[...]
    x3 = jnp.maximum(
        dinv * (pp_ref[0, :N] + pp_ref[1, :N] + h3s_ref[...]) + b3_ref[...], 0.0)
    xl = (jnp.dot(x1_ref[...], wa_ref[...], preferred_element_type=jnp.float32)
          + jnp.dot(x2_ref[...], wb_ref[...], preferred_element_type=jnp.float32)
          + jnp.dot(x3, wc_ref[...], preferred_element_type=jnp.float32)
          + bl1_ref[...])
    xls_ref[...] = dinv * xl


def _mm(a, b):
    return jnp.dot(a, b, preferred_element_type=jnp.float32)


def _head_masks():
    col = lax.broadcasted_iota(jnp.int32, (1, HID), 1)
    return [(col // HD == h).astype(jnp.float32) for h in range(NUM_HEADS)]


def _stack_heads(Qp, masks):
    return jnp.concatenate([Qp * m for m in masks], axis=0)


def _merge_heads(AV, nq):
    return jnp.concatenate(
        [AV[h * nq:(h + 1) * nq, h * HD:(h + 1) * HD]
         for h in range(NUM_HEADS)], axis=1)


def _attend_stacked(Qstack, Qp, K, V, nq):
    logits = lax.dot_general(Qstack, K, (((1,), (1,)), ((), ()))) * SCALE
    m = jnp.max(logits, axis=-1, keepdims=True)
    e = jnp.exp(logits - m)
    a = e / jnp.sum(e, axis=-1, keepdims=True)
    return Qp + _merge_heads(_mm(a, V), nq)


def _tc_attn(p4_ref, xls_ref, dinv_ref,
             wk1_ref, bk1_ref, wv1_ref, bv1_ref,
             s1_ref, wq1_ref, bq1_ref, wo1_ref, bo1_ref,
             wq2_ref, bq2_ref, wk2_ref, bk2_ref, wv2_ref, bv2_ref,
             wo2_ref, bo2_ref,
             s2_ref, wq3_ref, bq3_ref, wk3_ref, bk3_ref, wv3_ref, bv3_ref,
             wo3_ref, bo3_ref,
             wg_ref, bg_ref, wl1_ref, bl1_ref, wl2_ref, bl2_ref,
             out_ref):
    masks = _head_masks()
    dinv = dinv_ref[...]
    pxl = dinv * (p4_ref[0, :N] + p4_ref[1, :N] + xls_ref[...])
    K1 = _mm(pxl, wk1_ref[...]) + bk1_ref[...]
    V1 = _mm(pxl, wv1_ref[...]) + bv1_ref[...]

    Qp1 = _mm(s1_ref[...], wq1_ref[...]) + bq1_ref[...]
    Qs1 = _stack_heads(Qp1, masks)
    O1 = jnp.concatenate(
        [_attend_stacked(Qs1, Qp1, K1[g * NPG:(g + 1) * NPG],
                         V1[g * NPG:(g + 1) * NPG], NUM_SEEDS)
         for g in range(NUM_GRAPHS)], axis=0)
    X1 = O1 + jnp.maximum(_mm(O1, wo1_ref[...]) + bo1_ref[...], 0.0)

    Qp2 = _mm(X1, wq2_ref[...]) + bq2_ref[...]
    K2 = _mm(X1, wk2_ref[...]) + bk2_ref[...]
    V2 = _mm(X1, wv2_ref[...]) + bv2_ref[...]
    NS2 = NUM_SEEDS
    O2 = jnp.concatenate(
        [_attend_stacked(_stack_heads(Qp2[g * NS2:(g + 1) * NS2], masks),
                         Qp2[g * NS2:(g + 1) * NS2],
                         K2[g * NS2:(g + 1) * NS2],
                         V2[g * NS2:(g + 1) * NS2], NS2)
         for g in range(NUM_GRAPHS)], axis=0)
    X2 = O2 + jnp.maximum(_mm(O2, wo2_ref[...]) + bo2_ref[...], 0.0)

    Qp3 = _mm(s2_ref[...], wq3_ref[...]) + bq3_ref[...]
    Qs3 = _stack_heads(Qp3, masks)
    K3 = _mm(X2, wk3_ref[...]) + bk3_ref[...]
    V3 = _mm(X2, wv3_ref[...]) + bv3_ref[...]
    O3 = jnp.concatenate(
        [_attend_stacked(Qs3, Qp3, K3[g * NS2:(g + 1) * NS2],
                         V3[g * NS2:(g + 1) * NS2], 1)
         for g in range(NUM_GRAPHS)], axis=0)
    X3 = O3 + jnp.maximum(_mm(O3, wo3_ref[...]) + bo3_ref[...], 0.0)

    g = _mm(X3, wg_ref[...]) + bg_ref[...]
    h = jnp.maximum(_mm(g, wl1_ref[...]) + bl1_ref[...], 0.0)
    o = _mm(h, wl2_ref[...]) + bl2_ref[...]
    m = jnp.max(o, axis=-1, keepdims=True)
    lse = jnp.log(jnp.sum(jnp.exp(o - m), axis=-1, keepdims=True)) + m
    out_ref[...] = o - lse


def kernel(x, edge_index, batch, params):
    p = params
    f32 = jnp.float32

    src = edge_index[0].reshape(NW, EW)
    dst = edge_index[1].reshape(NW, EW)
    pad_s = jnp.zeros((NW, EWP - EW), jnp.int32)
    pad_d = jnp.full((NW, EWP - EW), N, jnp.int32)
    srcp = jnp.concatenate([src, pad_s], axis=1).reshape(NW, CH, CHUNK)
    dstp = jnp.concatenate([dst, pad_d], axis=1).reshape(NW, CH, CHUNK)

    ones_deg = jnp.ones((CHUNK, DEGW), f32)
    zeros_deg = jnp.zeros((RPT, DEGW), f32)
    zeros32 = jnp.zeros((RPT, 32), f32)
    zeros64 = jnp.zeros((RPT, 64), f32)

    degp = _sc_degree(dstp, ones_deg, zeros_deg)

    dinv, h1s = pl.pallas_call(
        _tc1,
        out_shape=[jax.ShapeDtypeStruct((N, 1), f32),
                   jax.ShapeDtypeStruct((N, 32), f32)],
    )(degp, x, p["conv1"]["W"])

    p1 = _sc_scatter32(h1s, srcp, dstp, zeros32)
    x1, h2s = pl.pallas_call(
        _tc_stage,
        out_shape=[jax.ShapeDtypeStruct((N, 32), f32),
                   jax.ShapeDtypeStruct((N, 32), f32)],
    )(p1, h1s, dinv, p["conv1"]["b"].reshape(1, 32), p["conv2"]["W"])

    p2 = _sc_scatter32(h2s, srcp, dstp, zeros32)
    x2, h3s = pl.pallas_call(
        _tc_stage,
        out_shape=[jax.ShapeDtypeStruct((N, 32), f32),
                   jax.ShapeDtypeStruct((N, 32), f32)],
    )(p2, h2s, dinv, p["conv2"]["b"].reshape(1, 32), p["conv3"]["W"])

    p3 = _sc_scatter32(h3s, srcp, dstp, zeros32)
    wl1 = p["gmt_lin1"]["W"]
    xls = pl.pallas_call(
        _tc4,
        out_shape=jax.ShapeDtypeStruct((N, HID), f32),
    )(p3, h3s, dinv, p["conv3"]["b"].reshape(1, 32), x1, x2,
      wl1[:32], wl1[32:64], wl1[64:], p["gmt_lin1"]["b"].reshape(1, HID))

    p4 = _sc_scatter64(xls, srcp, dstp, zeros64)

    def lin_args(*names):
        args = []
        for name in names:
            args += [p[name]["W"], p[name]["b"].reshape(1, -1)]
        return args

    s1 = p["S1"].reshape(NUM_SEEDS, HID)
    s2 = p["S2"].reshape(1, HID)

    out = pl.pallas_call(
        _tc_attn,
        out_shape=jax.ShapeDtypeStruct((NUM_GRAPHS, 10), f32),
    )(p4, xls, dinv,
      *lin_args("mab1_layer_k", "mab1_layer_v"),
      s1, *lin_args("mab1_fc_q"), *lin_args("mab1_fc_o"),
      *lin_args("mab2_fc_q", "mab2_layer_k", "mab2_layer_v", "mab2_fc_o"),
      s2, *lin_args("mab3_fc_q", "mab3_layer_k", "mab3_layer_v", "mab3_fc_o"),
      *lin_args("gmt_lin2", "lin1", "lin2"))

    return out


# speedup vs baseline: 22.8960x; 1.0256x over previous
import functools
import math

import jax
import jax.numpy as jnp
from jax import lax
from jax.experimental import pallas as pl
from jax.experimental.pallas import tpu as pltpu
from jax.experimental.pallas import tpu_sc as plsc

N = 10000
E = 320000
NUM_GRAPHS = 50
NPG = 200
HID = 64
NUM_HEADS = 4
HD = HID // NUM_HEADS
NUM_SEEDS = 50
SCALE = 1.0 / math.sqrt(HID)

NC = 2
NS = 16
NW = NC * NS
EW = E // NW
CHUNK = 128
MBUF = 8
LEAD = 4
CH = 80
EWP = CH * CHUNK
NPAD = 10112
RPT = NPAD // NS
DEGW = 16


def _sc_mesh():
    return plsc.VectorSubcoreMesh(core_axis_name="c", subcore_axis_name="s")


_SC_PARAMS = pltpu.CompilerParams(use_tc_tiling_on_sc=False)


@functools.partial(
    pl.kernel,
    out_type=jax.ShapeDtypeStruct((NC, NPAD, DEGW), jnp.float32),
    mesh=_sc_mesh(),
    compiler_params=_SC_PARAMS,
    scratch_types=[
        pltpu.VMEM((CH, CHUNK), jnp.int32),
        pltpu.VMEM((CHUNK, DEGW), jnp.float32),
        pltpu.VMEM_SHARED((NPAD, DEGW), jnp.float32),
        pltpu.SemaphoreType.DMA,
    ],
)
def _sc_degree(dstp_hbm, ones_hbm, zeros_hbm, out_hbm, dst_v, ones_v, acc,
               dsem):
    c = lax.axis_index("c")
    s = lax.axis_index("s")
    w = c * NS + s
    pltpu.sync_copy(zeros_hbm, acc.at[pl.ds(s * RPT, RPT)])
    pltpu.sync_copy(ones_hbm, ones_v)
    pltpu.sync_copy(dstp_hbm.at[w], dst_v)
    plsc.subcore_barrier()

    def body(j, carry):
        pltpu.async_copy(ones_v, acc.at[dst_v.at[j]], dsem, add=True)
        return carry

    lax.fori_loop(0, CH, body, 0)

    def drain(j, carry):
        pltpu.make_async_copy(ones_v, acc.at[dst_v.at[0]], dsem).wait()
        return carry

    lax.fori_loop(0, CH, drain, 0)
    plsc.subcore_barrier()
    pltpu.sync_copy(acc.at[pl.ds(s * RPT, RPT)],
                    out_hbm.at[c, pl.ds(s * RPT, RPT)])


def _make_sc_scatter(W):
    @functools.partial(
        pl.kernel,
        out_type=jax.ShapeDtypeStruct((NC, NPAD, W), jnp.float32),
        mesh=_sc_mesh(),
        compiler_params=_SC_PARAMS,
        scratch_types=[
            pltpu.VMEM((CH, CHUNK), jnp.int32),
            pltpu.VMEM((CH, CHUNK), jnp.int32),
            [pltpu.VMEM((CHUNK, W), jnp.float32) for _ in range(MBUF)],
            pltpu.VMEM_SHARED((NPAD, W), jnp.float32),
            [pltpu.SemaphoreType.DMA for _ in range(MBUF)],
            [pltpu.SemaphoreType.DMA for _ in range(MBUF)],
        ],
    )
    def k(hs_hbm, srcp_hbm, dstp_hbm, zeros_hbm, out_hbm,
          src_v, dst_v, rows, acc, gsem, ssem):
        c = lax.axis_index("c")
        s = lax.axis_index("s")
        w = c * NS + s
        pltpu.sync_copy(zeros_hbm, acc.at[pl.ds(s * RPT, RPT)])
        pltpu.sync_copy(srcp_hbm.at[w], src_v)
        pltpu.sync_copy(dstp_hbm.at[w], dst_v)
        for b in range(LEAD):
            pltpu.async_copy(hs_hbm.at[src_v.at[b]], rows[b], gsem[b])
        plsc.subcore_barrier()

        def body(i, carry):
            for b in range(MBUF):
                j = i * MBUF + b
                bg = (b + LEAD) % MBUF

                @pl.when(j + LEAD < CH)
                def _():
                    @pl.when(j + LEAD >= MBUF)
                    def _():
                        pltpu.make_async_copy(
                            rows[bg], acc.at[dst_v.at[0]], ssem[bg]).wait()
                    pltpu.async_copy(hs_hbm.at[src_v.at[j + LEAD]],
                                     rows[bg], gsem[bg])

                pltpu.make_async_copy(hs_hbm.at[src_v.at[j]],
                                      rows[b], gsem[b]).wait()
                pltpu.async_copy(rows[b], acc.at[dst_v.at[j]], ssem[b],
                                 add=True)
            return carry

        lax.fori_loop(0, CH // MBUF, body, 0)
        for b in range(MBUF):
            pltpu.make_async_copy(rows[b], acc.at[dst_v.at[0]],
                                  ssem[b]).wait()
        plsc.subcore_barrier()
        pltpu.sync_copy(acc.at[pl.ds(s * RPT, RPT)],
                        out_hbm.at[c, pl.ds(s * RPT, RPT)])

    return k


_sc_scatter32 = _make_sc_scatter(32)
_sc_scatter64 = _make_sc_scatter(64)


def _tc1(degp_ref, x_ref, w1_ref, dinv_ref, h1s_ref):
    d = degp_ref[0, :N, 0:1] + degp_ref[1, :N, 0:1] + 1.0
    dinv = lax.rsqrt(d)
    dinv_ref[...] = dinv
    h = jnp.dot(x_ref[...], w1_ref[...], preferred_element_type=jnp.float32)
    h1s_ref[...] = dinv * h


def _tc_stage(pp_ref, hs_ref, dinv_ref, b_ref, wn_ref, x_ref, hsn_ref):
    dinv = dinv_ref[...]
    xk = jnp.maximum(
        dinv * (pp_ref[0, :N] + pp_ref[1, :N] + hs_ref[...]) + b_ref[...], 0.0)
    x_ref[...] = xk
    hsn_ref[...] = dinv * jnp.dot(xk, wn_ref[...],
                                  preferred_element_type=jnp.float32)


def _tc4(pp_ref, h3s_ref, dinv_ref, b3_ref, x1_ref, x2_ref,
         wa_ref, wb_ref, wc_ref, bl1_ref, xls_ref):
    dinv = dinv_ref[...]
    x3 = jnp.maximum(
        dinv * (pp_ref[0, :N] + pp_ref[1, :N] + h3s_ref[...]) + b3_ref[...], 0.0)
    xl = (jnp.dot(x1_ref[...], wa_ref[...], preferred_element_type=jnp.float32)
          + jnp.dot(x2_ref[...], wb_ref[...], preferred_element_type=jnp.float32)
          + jnp.dot(x3, wc_ref[...], preferred_element_type=jnp.float32)
          + bl1_ref[...])
    xls_ref[...] = dinv * xl


def _mm(a, b):
    return jnp.dot(a, b, preferred_element_type=jnp.float32)


def _head_masks():
    col = lax.broadcasted_iota(jnp.int32, (1, HID), 1)
    return [(col // HD == h).astype(jnp.float32) for h in range(NUM_HEADS)]


def _stack_heads(Qp, masks):
    return jnp.concatenate([Qp * m for m in masks], axis=0)


def _merge_heads3(O, nq):
    return jnp.concatenate(
        [O[:, h * nq:(h + 1) * nq, h * HD:(h + 1) * HD]
         for h in range(NUM_HEADS)], axis=2)


def _softmax_last(logits):
    m = jnp.max(logits, axis=-1, keepdims=True)
    e = jnp.exp(logits - m)
    return e / jnp.sum(e, axis=-1, keepdims=True)


def _batched_attend(Qs3, K3, V3, nq):
    logits = lax.dot_general(
        Qs3, K3, (((2,), (2,)), ((0,), (0,)))) * SCALE
    a = _softmax_last(logits)
    O = lax.dot_general(a, V3, (((2,), (1,)), ((0,), (0,))))
    return _merge_heads3(O, nq)


def _tc_attn(p4_ref, xls_ref, dinv_ref,
             wk1_ref, bk1_ref, wv1_ref, bv1_ref,
             s1_ref, wq1_ref, bq1_ref, wo1_ref, bo1_ref,
             wq2_ref, bq2_ref, wk2_ref, bk2_ref, wv2_ref, bv2_ref,
             wo2_ref, bo2_ref,
             s2_ref, wq3_ref, bq3_ref, wk3_ref, bk3_ref, wv3_ref, bv3_ref,
             wo3_ref, bo3_ref,
             wg_ref, bg_ref, wl1_ref, bl1_ref, wl2_ref, bl2_ref,
             out_ref):
    masks = _head_masks()
    dinv = dinv_ref[...]
    pxl = dinv * (p4_ref[0, :N] + p4_ref[1, :N] + xls_ref[...])
    K1 = _mm(pxl, wk1_ref[...]) + bk1_ref[...]
    V1 = _mm(pxl, wv1_ref[...]) + bv1_ref[...]

    Qp1 = _mm(s1_ref[...], wq1_ref[...]) + bq1_ref[...]
    Qs1 = _stack_heads(Qp1, masks)
    logitsT = lax.dot_general(K1, Qs1, (((1,), (1,)), ((), ()))) * SCALE
    L = logitsT.reshape(NUM_GRAPHS, NPG, NUM_HEADS * NUM_SEEDS)
    m1 = jnp.max(L, axis=1, keepdims=True)
    e1 = jnp.exp(L - m1)
    a1 = e1 / jnp.sum(e1, axis=1, keepdims=True)
    AV1 = lax.dot_general(a1, V1.reshape(NUM_GRAPHS, NPG, HID),
                          (((1,), (1,)), ((0,), (0,))))
    O1 = (Qp1[None] + _merge_heads3(AV1, NUM_SEEDS)).reshape(
        NUM_GRAPHS * NUM_SEEDS, HID)
    X1 = O1 + jnp.maximum(_mm(O1, wo1_ref[...]) + bo1_ref[...], 0.0)

    Qp2 = _mm(X1, wq2_ref[...]) + bq2_ref[...]
    K2 = _mm(X1, wk2_ref[...]) + bk2_ref[...]
    V2 = _mm(X1, wv2_ref[...]) + bv2_ref[...]
    Qp2_3 = Qp2.reshape(NUM_GRAPHS, NUM_SEEDS, HID)
    Qs2 = jnp.concatenate([Qp2_3 * m for m in masks], axis=1)
    O2 = (Qp2_3 + _batched_attend(
        Qs2, K2.reshape(NUM_GRAPHS, NUM_SEEDS, HID),
        V2.reshape(NUM_GRAPHS, NUM_SEEDS, HID), NUM_SEEDS)).reshape(
            NUM_GRAPHS * NUM_SEEDS, HID)
    X2 = O2 + jnp.maximum(_mm(O2, wo2_ref[...]) + bo2_ref[...], 0.0)

    Qp3 = _mm(s2_ref[...], wq3_ref[...]) + bq3_ref[...]
    Qs3 = _stack_heads(Qp3, masks)
    K3 = _mm(X2, wk3_ref[...]) + bk3_ref[...]
    V3 = _mm(X2, wv3_ref[...]) + bv3_ref[...]
    O3 = (Qp3[None] + _batched_attend(
        jnp.broadcast_to(Qs3[None], (NUM_GRAPHS, NUM_HEADS, HID)),
        K3.reshape(NUM_GRAPHS, NUM_SEEDS, HID),
        V3.reshape(NUM_GRAPHS, NUM_SEEDS, HID), 1)).reshape(
            NUM_GRAPHS, HID)
    X3 = O3 + jnp.maximum(_mm(O3, wo3_ref[...]) + bo3_ref[...], 0.0)

    g = _mm(X3, wg_ref[...]) + bg_ref[...]
    h = jnp.maximum(_mm(g, wl1_ref[...]) + bl1_ref[...], 0.0)
    o = _mm(h, wl2_ref[...]) + bl2_ref[...]
    m = jnp.max(o, axis=-1, keepdims=True)
    lse = jnp.log(jnp.sum(jnp.exp(o - m), axis=-1, keepdims=True)) + m
    out_ref[...] = o - lse


def kernel(x, edge_index, batch, params):
    p = params
    f32 = jnp.float32

    src = edge_index[0].reshape(NW, EW)
    dst = edge_index[1].reshape(NW, EW)
    pad_s = jnp.zeros((NW, EWP - EW), jnp.int32)
    pad_d = jnp.full((NW, EWP - EW), N, jnp.int32)
    srcp = jnp.concatenate([src, pad_s], axis=1).reshape(NW, CH, CHUNK)
    dstp = jnp.concatenate([dst, pad_d], axis=1).reshape(NW, CH, CHUNK)

    ones_deg = jnp.ones((CHUNK, DEGW), f32)
    zeros_deg = jnp.zeros((RPT, DEGW), f32)
    zeros32 = jnp.zeros((RPT, 32), f32)
    zeros64 = jnp.zeros((RPT, 64), f32)

    degp = _sc_degree(dstp, ones_deg, zeros_deg)

    dinv, h1s = pl.pallas_call(
        _tc1,
        out_shape=[jax.ShapeDtypeStruct((N, 1), f32),
                   jax.ShapeDtypeStruct((N, 32), f32)],
    )(degp, x, p["conv1"]["W"])

    p1 = _sc_scatter32(h1s, srcp, dstp, zeros32)
    x1, h2s = pl.pallas_call(
        _tc_stage,
        out_shape=[jax.ShapeDtypeStruct((N, 32), f32),
                   jax.ShapeDtypeStruct((N, 32), f32)],
    )(p1, h1s, dinv, p["conv1"]["b"].reshape(1, 32), p["conv2"]["W"])

    p2 = _sc_scatter32(h2s, srcp, dstp, zeros32)
    x2, h3s = pl.pallas_call(
        _tc_stage,
        out_shape=[jax.ShapeDtypeStruct((N, 32), f32),
                   jax.ShapeDtypeStruct((N, 32), f32)],
    )(p2, h2s, dinv, p["conv2"]["b"].reshape(1, 32), p["conv3"]["W"])

    p3 = _sc_scatter32(h3s, srcp, dstp, zeros32)
    wl1 = p["gmt_lin1"]["W"]
    xls = pl.pallas_call(
        _tc4,
        out_shape=jax.ShapeDtypeStruct((N, HID), f32),
    )(p3, h3s, dinv, p["conv3"]["b"].reshape(1, 32), x1, x2,
      wl1[:32], wl1[32:64], wl1[64:], p["gmt_lin1"]["b"].reshape(1, HID))

    p4 = _sc_scatter64(xls, srcp, dstp, zeros64)

    def lin_args(*names):
        args = []
        for name in names:
            args += [p[name]["W"], p[name]["b"].reshape(1, -1)]
        return args

    s1 = p["S1"].reshape(NUM_SEEDS, HID)
    s2 = p["S2"].reshape(1, HID)

    out = pl.pallas_call(
        _tc_attn,
        out_shape=jax.ShapeDtypeStruct((NUM_GRAPHS, 10), f32),
    )(p4, xls, dinv,
      *lin_args("mab1_layer_k", "mab1_layer_v"),
      s1, *lin_args("mab1_fc_q"), *lin_args("mab1_fc_o"),
      *lin_args("mab2_fc_q", "mab2_layer_k", "mab2_layer_v", "mab2_fc_o"),
      s2, *lin_args("mab3_fc_q", "mab3_layer_k", "mab3_layer_v", "mab3_fc_o"),
      *lin_args("gmt_lin2", "lin1", "lin2"))

    return out
